# baseline (speedup 1.0000x reference)
import functools

import jax
import jax.numpy as jnp
from jax import lax
from jax.experimental import pallas as pl
from jax.experimental.pallas import tpu as pltpu
from jax.experimental.pallas import tpu_sc as plsc

NC = 2
NS = 16
NW = NC * NS
DP = 128
BB = 256
L = 16


def _emb_body(units_per_w, n_bblk, D, tok_hbm, table_hbm, out_hbm,
              idx_v, ga, gb, xa, xb, gsem, ssem):
    wid = lax.axis_index("s") * NC + lax.axis_index("c")
    ubase = wid * units_per_w
    pltpu.sync_copy(tok_hbm.at[wid], idx_v)

    gbuf = (ga, gb)
    xbuf = (xa, xb)
    n_str = BB // 128

    def fire_gather(h, p):
        for j in range(n_str):
            pltpu.async_copy(
                table_hbm.at[idx_v.at[h * n_str + j]],
                gbuf[p].at[pl.ds(j * 128, 128)],
                gsem.at[p])

    def drain_gather(p):
        for j in range(n_str):
            pltpu.make_async_copy(
                table_hbm.at[pl.ds(0, 128)],
                gbuf[p].at[pl.ds(j * 128, 128)],
                gsem.at[p]).wait()

    def fire_store(h, p):
        u = ubase + h
        s = u // n_bblk
        b0 = (u % n_bblk) * BB
        pltpu.async_copy(
            xbuf[p], out_hbm.at[s, :, pl.ds(b0, BB)], ssem.at[p])

    def wait_store(p):
        pltpu.make_async_copy(
            xbuf[p], out_hbm.at[0, :, pl.ds(0, BB)], ssem.at[p]).wait()

    iota = lax.iota(jnp.int32, L)
    rowvecs = [iota + (c0 * L) for c0 in range(BB // L)]

    def transpose(p):
        g = gbuf[p]
        x = xbuf[p]

        @pl.loop(0, D)
        def _(d):
            dvec = lax.broadcast_in_dim(d, (L,), ())
            for c0 in range(BB // L):
                x[d, pl.ds(c0 * L, L)] = plsc.load_gather(
                    g, [rowvecs[c0], dvec])

    fire_gather(0, 0)

    @pl.loop(0, units_per_w, step=2)
    def _(h):
        fire_gather(h + 1, 1)
        drain_gather(0)

        @pl.when(h >= 2)
        def _():
            wait_store(0)
        transpose(0)
        fire_store(h, 0)

        @pl.when(h + 2 < units_per_w)
        def _():
            fire_gather(h + 2, 0)
        drain_gather(1)

        @pl.when(h >= 1)
        def _():
            wait_store(1)
        transpose(1)
        fire_store(h + 1, 1)

    wait_store(0)
    wait_store(1)


def kernel(token_ids, E):
    B, S = token_ids.shape
    V, D = E.shape

    n_bblk = B // BB
    n_units = S * n_bblk
    units_per_w = n_units // NW
    assert n_bblk * BB == B and units_per_w * NW == n_units
    assert units_per_w % 2 == 0

    tok = token_ids.T.reshape(NW, units_per_w * (BB // 128), 128).astype(
        jnp.int32)
    Ep = jnp.pad(E, ((0, 0), (0, DP - D)))

    mesh = plsc.VectorSubcoreMesh(
        core_axis_name="c", subcore_axis_name="s", num_cores=NC,
        num_subcores=NS)

    run = functools.partial(
        pl.kernel,
        out_type=jax.ShapeDtypeStruct((S, D, B), jnp.float32),
        mesh=mesh,
        compiler_params=pltpu.CompilerParams(
            use_tc_tiling_on_sc=True, needs_layout_passes=False),
        scratch_types=[
            pltpu.VMEM((units_per_w * (BB // 128), 128), jnp.int32),
            pltpu.VMEM((BB, DP), jnp.float32),
            pltpu.VMEM((BB, DP), jnp.float32),
            pltpu.VMEM((D, BB), jnp.float32),
            pltpu.VMEM((D, BB), jnp.float32),
            pltpu.SemaphoreType.DMA((2,)),
            pltpu.SemaphoreType.DMA((2,)),
        ],
    )(functools.partial(_emb_body, units_per_w, n_bblk, D))

    x = run(tok, Ep)
    return jnp.transpose(x, (2, 0, 1))


# speedup vs baseline: 1.1499x; 1.1499x over previous
import functools

import jax
import jax.numpy as jnp
from jax import lax
from jax.experimental import pallas as pl
from jax.experimental.pallas import tpu as pltpu
from jax.experimental.pallas import tpu_sc as plsc

NC = 2
NS = 16
NW = NC * NS
DP = 128
BB = 256
L = 16


def _emb_body(units_per_w, n_bblk, D, tok_hbm, table_hbm, out_hbm,
              idx_v, ga, gb, xa, xb, gsem, ssem):
    wid = lax.axis_index("s") * NC + lax.axis_index("c")
    ubase = wid * units_per_w
    pltpu.sync_copy(tok_hbm.at[wid], idx_v)

    gbuf = (ga, gb)
    xbuf = (xa, xb)
    n_str = BB // 128

    def fire_gather(h, p):
        for j in range(n_str):
            pltpu.async_copy(
                table_hbm.at[idx_v.at[h * n_str + j]],
                gbuf[p].at[pl.ds(j * 128, 128)],
                gsem.at[p])

    def drain_gather(p):
        for j in range(n_str):
            pltpu.make_async_copy(
                table_hbm.at[pl.ds(0, 128)],
                gbuf[p].at[pl.ds(j * 128, 128)],
                gsem.at[p]).wait()

    def fire_store(h, p):
        u = ubase + h
        s = u // n_bblk
        b0 = (u % n_bblk) * BB
        pltpu.async_copy(
            xbuf[p], out_hbm.at[s, :, pl.ds(b0, BB)], ssem.at[p])

    def wait_store(p):
        pltpu.make_async_copy(
            xbuf[p], out_hbm.at[0, :, pl.ds(0, BB)], ssem.at[p]).wait()

    iota = lax.iota(jnp.int32, L)
    dvecs = [iota + (d0 * L) for d0 in range(D // L)]

    def transpose(p):
        g = gbuf[p]
        x = xbuf[p]

        @pl.loop(0, BB)
        def _(c):
            cvec = lax.broadcast_in_dim(c, (L,), ())
            for d0 in range(D // L):
                plsc.store_scatter(
                    x, [dvecs[d0], cvec], g[c, pl.ds(d0 * L, L)])

    fire_gather(0, 0)

    @pl.loop(0, units_per_w, step=2)
    def _(h):
        fire_gather(h + 1, 1)
        drain_gather(0)

        @pl.when(h >= 2)
        def _():
            wait_store(0)
        transpose(0)
        fire_store(h, 0)

        @pl.when(h + 2 < units_per_w)
        def _():
            fire_gather(h + 2, 0)
        drain_gather(1)

        @pl.when(h >= 1)
        def _():
            wait_store(1)
        transpose(1)
        fire_store(h + 1, 1)

    wait_store(0)
    wait_store(1)


def kernel(token_ids, E):
    B, S = token_ids.shape
    V, D = E.shape

    n_bblk = B // BB
    n_units = S * n_bblk
    units_per_w = n_units // NW
    assert n_bblk * BB == B and units_per_w * NW == n_units
    assert units_per_w % 2 == 0

    tok = token_ids.T.reshape(NW, units_per_w * (BB // 128), 128).astype(
        jnp.int32)
    Ep = jnp.pad(E, ((0, 0), (0, DP - D)))

    mesh = plsc.VectorSubcoreMesh(
        core_axis_name="c", subcore_axis_name="s", num_cores=NC,
        num_subcores=NS)

    run = functools.partial(
        pl.kernel,
        out_type=jax.ShapeDtypeStruct((S, D, B), jnp.float32),
        mesh=mesh,
        compiler_params=pltpu.CompilerParams(
            use_tc_tiling_on_sc=True, needs_layout_passes=False),
        scratch_types=[
            pltpu.VMEM((units_per_w * (BB // 128), 128), jnp.int32),
            pltpu.VMEM((BB, DP), jnp.float32),
            pltpu.VMEM((BB, DP), jnp.float32),
            pltpu.VMEM((D, BB), jnp.float32),
            pltpu.VMEM((D, BB), jnp.float32),
            pltpu.SemaphoreType.DMA((2,)),
            pltpu.SemaphoreType.DMA((2,)),
        ],
    )(functools.partial(_emb_body, units_per_w, n_bblk, D))

    x = run(tok, Ep)
    return jnp.transpose(x, (2, 0, 1))


# speedup vs baseline: 1.1804x; 1.0266x over previous
import functools

import jax
import jax.numpy as jnp
from jax import lax
from jax.experimental import pallas as pl
from jax.experimental.pallas import tpu as pltpu
from jax.experimental.pallas import tpu_sc as plsc

NC = 2
NS = 16
NW = NC * NS
DP = 128
BB = 256
L = 16


def _emb_body(units_per_w, n_bblk, D, tok_hbm, table_hbm, out_hbm,
              idx_v, ga, gb, xa, xb, gsem, ssem):
    wid = lax.axis_index("s") * NC + lax.axis_index("c")
    ubase = wid * units_per_w
    pltpu.sync_copy(tok_hbm.at[wid], idx_v)

    gbuf = (ga, gb)
    xbuf = (xa, xb)
    n_str = BB // 128

    def fire_gather(h, p):
        for j in range(n_str):
            pltpu.async_copy(
                table_hbm.at[idx_v.at[h * n_str + j]],
                gbuf[p].at[pl.ds(j * 128, 128)],
                gsem.at[p])

    def drain_gather(p):
        for j in range(n_str):
            pltpu.make_async_copy(
                table_hbm.at[pl.ds(0, 128)],
                gbuf[p].at[pl.ds(j * 128, 128)],
                gsem.at[p]).wait()

    def fire_store(h, p):
        u = ubase + h
        s = u // n_bblk
        b0 = (u % n_bblk) * BB
        pltpu.async_copy(
            xbuf[p], out_hbm.at[s, :, pl.ds(b0, BB)], ssem.at[p])

    def wait_store(p):
        pltpu.make_async_copy(
            xbuf[p], out_hbm.at[0, :, pl.ds(0, BB)], ssem.at[p]).wait()

    iota = lax.iota(jnp.int32, L)
    dvecs = [iota + (d0 * L) for d0 in range(D // L)]

    def transpose(p):
        g = gbuf[p]
        x = xbuf[p]

        @pl.loop(0, BB, step=4)
        def _(c):
            vals = []
            for dc in range(4):
                cvec = lax.broadcast_in_dim(c + dc, (L,), ())
                for d0 in range(D // L):
                    vals.append((d0, cvec, g[c + dc, pl.ds(d0 * L, L)]))
            for d0, cvec, v in vals:
                plsc.store_scatter(x, [dvecs[d0], cvec], v)

    fire_gather(0, 0)

    @pl.loop(0, units_per_w, step=2)
    def _(h):
        fire_gather(h + 1, 1)
        drain_gather(0)

        @pl.when(h >= 2)
        def _():
            wait_store(0)
        transpose(0)
        fire_store(h, 0)

        @pl.when(h + 2 < units_per_w)
        def _():
            fire_gather(h + 2, 0)
        drain_gather(1)

        @pl.when(h >= 1)
        def _():
            wait_store(1)
        transpose(1)
        fire_store(h + 1, 1)

    wait_store(0)
    wait_store(1)


def kernel(token_ids, E):
    B, S = token_ids.shape
    V, D = E.shape

    n_bblk = B // BB
    n_units = S * n_bblk
    units_per_w = n_units // NW
    assert n_bblk * BB == B and units_per_w * NW == n_units
    assert units_per_w % 2 == 0

    tok = token_ids.T.reshape(NW, units_per_w * (BB // 128), 128).astype(
        jnp.int32)
    Ep = jnp.pad(E, ((0, 0), (0, DP - D)))

    mesh = plsc.VectorSubcoreMesh(
        core_axis_name="c", subcore_axis_name="s", num_cores=NC,
        num_subcores=NS)

    run = functools.partial(
        pl.kernel,
        out_type=jax.ShapeDtypeStruct((S, D, B), jnp.float32),
        mesh=mesh,
        compiler_params=pltpu.CompilerParams(
            use_tc_tiling_on_sc=True, needs_layout_passes=False),
        scratch_types=[
            pltpu.VMEM((units_per_w * (BB // 128), 128), jnp.int32),
            pltpu.VMEM((BB, DP), jnp.float32),
            pltpu.VMEM((BB, DP), jnp.float32),
            pltpu.VMEM((D, BB), jnp.float32),
            pltpu.VMEM((D, BB), jnp.float32),
            pltpu.SemaphoreType.DMA((2,)),
            pltpu.SemaphoreType.DMA((2,)),
        ],
    )(functools.partial(_emb_body, units_per_w, n_bblk, D))

    x = run(tok, Ep)
    return jnp.transpose(x, (2, 0, 1))


# speedup vs baseline: 1.4317x; 1.2129x over previous
import functools

import jax
import jax.numpy as jnp
from jax import lax
from jax.experimental import pallas as pl
from jax.experimental.pallas import tpu as pltpu
from jax.experimental.pallas import tpu_sc as plsc

NC = 2
NS = 16
NW = NC * NS
BPG = 2


def _emb_body(batches_per_w, S, D, token_hbm, table_hbm, out_hbm,
              idx_v, rows_v, gsem, ssem):
    n_groups = batches_per_w // BPG
    wid = lax.axis_index("s") * NC + lax.axis_index("c")
    pltpu.sync_copy(token_hbm.at[wid], idx_v)

    def fire_gathers(h, b):
        for j in range(BPG):
            pltpu.async_copy(
                table_hbm.at[idx_v.at[h * BPG + j]],
                rows_v.at[b, j],
                gsem.at[b])

    def drain_gathers(b):
        for j in range(BPG):
            pltpu.make_async_copy(
                table_hbm.at[pl.ds(0, S)], rows_v.at[b, j], gsem.at[b]).wait()

    def fire_store(h, b):
        pltpu.async_copy(
            rows_v.at[b], out_hbm.at[pl.ds(wid * batches_per_w + h * BPG, BPG)],
            ssem.at[b])

    def wait_store(b):
        pltpu.make_async_copy(
            rows_v.at[b], out_hbm.at[pl.ds(0, BPG)], ssem.at[b]).wait()

    fire_gathers(0, 0)

    @pl.loop(0, n_groups, step=2)
    def _(g):
        @pl.when(g >= 1)
        def _():
            wait_store(1)
        fire_gathers(g + 1, 1)
        drain_gathers(0)
        fire_store(g, 0)

        wait_store(0)

        @pl.when(g + 2 < n_groups)
        def _():
            fire_gathers(g + 2, 0)
        drain_gathers(1)
        fire_store(g + 1, 1)

    wait_store(1)


def kernel(token_ids, E):
    B, S = token_ids.shape
    V, D = E.shape

    batches_per_w = B // NW
    assert batches_per_w * NW == B and batches_per_w % BPG == 0
    assert (batches_per_w // BPG) % 2 == 0

    tok = token_ids.reshape(NW, batches_per_w, S).astype(jnp.int32)

    mesh = plsc.VectorSubcoreMesh(
        core_axis_name="c", subcore_axis_name="s", num_cores=NC,
        num_subcores=NS)

    run = functools.partial(
        pl.kernel,
        out_type=jax.ShapeDtypeStruct((B, S, D), jnp.float32),
        mesh=mesh,
        compiler_params=pltpu.CompilerParams(use_tc_tiling_on_sc=False),
        scratch_types=[
            pltpu.VMEM((batches_per_w, S), jnp.int32),
            pltpu.VMEM((2, BPG, S, D), jnp.float32),
            pltpu.SemaphoreType.DMA((2,)),
            pltpu.SemaphoreType.DMA((2,)),
        ],
    )(functools.partial(_emb_body, batches_per_w, S, D))

    return run(tok, E)
